# Initial kernel scaffold; baseline (speedup 1.0000x reference)
import functools

import jax
import jax.numpy as jnp
from jax import lax
from jax.experimental import pallas as pl
from jax.experimental.pallas import tpu as pltpu
from jax.experimental.pallas import tpu_sc as plsc

_ALPHA = 0.8
_N_ROWS = 16384
_N_COLS = 128
_N_IDX = 8192

_ROW_BLK = 1024
_GRID = _N_ROWS // _ROW_BLK

_NC = 2
_NS = 16
_NW = _NC * _NS
_IDX_PER_TILE = _N_IDX // _NW
_LANES = 16


def _rowsum_body(inp_ref, tgt_ref, out_ref):
    d = inp_ref[...] - tgt_ref[...]
    out_ref[...] = jnp.sum(d * d, axis=1)[None, :]


_rowsum_call = pl.pallas_call(
    _rowsum_body,
    grid=(_GRID,),
    in_specs=[
        pl.BlockSpec((_ROW_BLK, _N_COLS), lambda i: (i, 0)),
        pl.BlockSpec((_ROW_BLK, _N_COLS), lambda i: (i, 0)),
    ],
    out_specs=pl.BlockSpec((1, _ROW_BLK), lambda i: (i, 0)),
    out_shape=jax.ShapeDtypeStruct((_GRID, _ROW_BLK), jnp.float32),
)


def _sc_gather_body(rowsum_hbm, one_hbm, zero_hbm, out_hbm,
                    table_v, one_v, zero_v, out_v):
    cid = lax.axis_index("c")
    sid = lax.axis_index("s")
    wid = sid * _NC + cid
    base = wid * _IDX_PER_TILE

    pltpu.sync_copy(rowsum_hbm, table_v)
    pltpu.sync_copy(one_hbm.at[pl.ds(base, _IDX_PER_TILE)], one_v)
    pltpu.sync_copy(zero_hbm.at[pl.ds(base, _IDX_PER_TILE)], zero_v)

    acc1 = jnp.zeros((_LANES,), jnp.float32)
    acc0 = jnp.zeros((_LANES,), jnp.float32)
    for i in range(_IDX_PER_TILE // _LANES):
        i1 = one_v[pl.ds(i * _LANES, _LANES)]
        i0 = zero_v[pl.ds(i * _LANES, _LANES)]
        acc1 = acc1 + plsc.load_gather(table_v, [i1])
        acc0 = acc0 + plsc.load_gather(table_v, [i0])
    acc = jnp.float32(1.0 - _ALPHA) * acc1 + jnp.float32(_ALPHA) * acc0
    total = jnp.sum(acc)
    out_v[...] = jnp.full((_LANES,), total, jnp.float32)
    pltpu.sync_copy(out_v, out_hbm.at[wid])


_sc_gather_call = functools.partial(
    pl.kernel,
    mesh=plsc.VectorSubcoreMesh(core_axis_name="c", subcore_axis_name="s"),
    out_type=jax.ShapeDtypeStruct((_NW, _LANES), jnp.float32),
    scratch_types=[
        pltpu.VMEM((_N_ROWS,), jnp.float32),
        pltpu.VMEM((_IDX_PER_TILE,), jnp.int32),
        pltpu.VMEM((_IDX_PER_TILE,), jnp.int32),
        pltpu.VMEM((_LANES,), jnp.float32),
    ],
)(_sc_gather_body)


def kernel(one_index, zero_index, target, input):
    rowsum = _rowsum_call(input, target).reshape(_N_ROWS)
    partials = _sc_gather_call(rowsum, one_index, zero_index)
    return jnp.sum(partials[:, 0])


# speedup vs baseline: 1.2430x; 1.2430x over previous
import functools

import jax
import jax.numpy as jnp
from jax import lax
from jax.experimental import pallas as pl
from jax.experimental.pallas import tpu as pltpu
from jax.experimental.pallas import tpu_sc as plsc

_ALPHA = 0.8
_N_ROWS = 16384
_N_COLS = 128
_N_IDX = 8192

_ROW_BLK = 1024
_GRID = _N_ROWS // _ROW_BLK

_NC = 2
_NS = 16
_NW = _NC * _NS
_IDX_PER_TILE = _N_IDX // _NW
_LANES = 16


def _rowsum_body(inp_ref, tgt_ref, out_ref):
    d = inp_ref[...] - tgt_ref[...]
    out_ref[...] = jnp.sum(d * d, axis=1)[None, None, :]


_rowsum_call = pl.pallas_call(
    _rowsum_body,
    grid=(_GRID,),
    in_specs=[
        pl.BlockSpec((_ROW_BLK, _N_COLS), lambda i: (i, 0)),
        pl.BlockSpec((_ROW_BLK, _N_COLS), lambda i: (i, 0)),
    ],
    out_specs=pl.BlockSpec((1, 1, _ROW_BLK), lambda i: (i, 0, 0)),
    out_shape=jax.ShapeDtypeStruct((_GRID, 1, _ROW_BLK), jnp.float32),
)


def _sc_gather_body(rowsum_hbm, one_hbm, zero_hbm, out_hbm,
                    table_v, one_v, zero_v, out_v):
    cid = lax.axis_index("c")
    sid = lax.axis_index("s")
    wid = sid * _NC + cid
    base = wid * _IDX_PER_TILE

    pltpu.sync_copy(rowsum_hbm, table_v)
    pltpu.sync_copy(one_hbm.at[pl.ds(base, _IDX_PER_TILE)], one_v)
    pltpu.sync_copy(zero_hbm.at[pl.ds(base, _IDX_PER_TILE)], zero_v)

    acc1 = jnp.zeros((_LANES,), jnp.float32)
    acc0 = jnp.zeros((_LANES,), jnp.float32)
    for i in range(_IDX_PER_TILE // _LANES):
        i1 = one_v[pl.ds(i * _LANES, _LANES)]
        i0 = zero_v[pl.ds(i * _LANES, _LANES)]
        acc1 = acc1 + plsc.load_gather(table_v, [i1])
        acc0 = acc0 + plsc.load_gather(table_v, [i0])
    acc = jnp.float32(1.0 - _ALPHA) * acc1 + jnp.float32(_ALPHA) * acc0
    total = jnp.sum(acc)
    out_v[...] = jnp.full((_LANES,), total, jnp.float32)
    pltpu.sync_copy(out_v, out_hbm.at[wid])


_sc_gather_call = functools.partial(
    pl.kernel,
    mesh=plsc.VectorSubcoreMesh(core_axis_name="c", subcore_axis_name="s"),
    out_type=jax.ShapeDtypeStruct((_NW, _LANES), jnp.float32),
    scratch_types=[
        pltpu.VMEM((_N_ROWS,), jnp.float32),
        pltpu.VMEM((_IDX_PER_TILE,), jnp.int32),
        pltpu.VMEM((_IDX_PER_TILE,), jnp.int32),
        pltpu.VMEM((_LANES,), jnp.float32),
    ],
    compiler_params=pltpu.CompilerParams(needs_layout_passes=False),
)(_sc_gather_body)


def kernel(one_index, zero_index, target, input):
    rowsum = _rowsum_call(input, target).reshape(_N_ROWS)
    partials = _sc_gather_call(rowsum, one_index, zero_index)
    return jnp.sum(partials[:, 0])
